# baseline (speedup 1.0000x reference)
import functools

import jax
import jax.numpy as jnp
from jax import lax
from jax.experimental import pallas as pl
from jax.experimental.pallas import tpu as pltpu
from jax.experimental.pallas import tpu_sc as plsc

B, T, D = 1024, 200, 64
NC, NS = 2, 16
NW = NC * NS
BB = 128
NB = B // BB
TR = T // (NW // NB)
LANES = 16

_mesh = plsc.VectorSubcoreMesh(core_axis_name="c", subcore_axis_name="s")


@functools.partial(
    pl.kernel,
    out_type=jax.ShapeDtypeStruct((T, D // 8, NB, 8, BB), jnp.float32),
    mesh=_mesh,
    compiler_params=pltpu.CompilerParams(
        use_tc_tiling_on_sc=False, needs_layout_passes=False),
    scratch_types=[
        pltpu.VMEM((TR, BB), jnp.int32),
        pltpu.VMEM((2, BB, D), jnp.float32),
        pltpu.VMEM((2, D // 8, 8, BB), jnp.float32),
        pltpu.VMEM((T, D), jnp.float32),
        pltpu.SemaphoreType.DMA,
        pltpu.SemaphoreType.DMA,
    ],
)
def _embed(xT_hbm, tab_hbm, pos_hbm, p_hbm, idx_v, rows_v, stage_v,
           pos_v, gsem, osem):
    wid = lax.axis_index("s") * NC + lax.axis_index("c")
    tb = lax.rem(wid, NB)
    t0 = lax.div(wid, NB) * TR
    pltpu.sync_copy(pos_hbm, pos_v)
    pltpu.sync_copy(
        xT_hbm.at[pl.ds(t0, TR), pl.ds(tb * BB, BB)], idx_v)

    def start_gather(tl, buf):
        pltpu.async_copy(tab_hbm.at[idx_v.at[tl]], rows_v.at[buf], gsem)

    def wait_gather(tl, buf):
        pltpu.make_async_copy(
            tab_hbm.at[idx_v.at[tl]], rows_v.at[buf], gsem).wait()

    start_gather(0, 0)
    lanes = lax.iota(jnp.int32, LANES)

    def t_body(tl, carry):
        t = t0 + tl
        buf = lax.rem(tl, 2)
        nbuf = lax.rem(tl + 1, 2)

        @pl.when(tl >= 1)
        def _():
            pltpu.make_async_copy(
                stage_v.at[nbuf], p_hbm.at[t - 1, :, tb], osem).wait()

        @pl.when(tl + 1 < TR)
        def _():
            start_gather(tl + 1, nbuf)

        bufs = jnp.full((LANES,), buf, jnp.int32)
        for d0 in range(0, D, LANES):
            pos_row = pos_v[t, pl.ds(d0, LANES)]
            for j in range(LANES):
                pv = jax.lax.broadcast(pos_row[j], (LANES,))
                d = d0 + j
                for g in range(8):
                    stage_v[buf, d // 8, d % 8, pl.ds(g * LANES, LANES)] = pv

        wait_gather(tl, buf)

        for d0 in range(0, D, LANES):

            def _kbody(k, c2):
                rot = lax.rem(lanes + k, LANES)
                rot_hi = lax.shift_right_logical(rot, 3) + (d0 // 8)
                rot_lo = lax.bitwise_and(rot, 7)
                dcol = rot + d0
                for b0 in range(0, BB, LANES):
                    vec = plsc.load_gather(
                        rows_v, [bufs, lanes + b0, dcol])
                    plsc.addupdate_scatter(
                        stage_v, [bufs, rot_hi, rot_lo, lanes + b0], vec)
                return c2

            lax.fori_loop(0, LANES, _kbody, 0, unroll=4)

        pltpu.async_copy(stage_v.at[buf], p_hbm.at[t, :, tb], osem)
        return carry

    lax.fori_loop(0, TR, t_body, 0)
    pltpu.make_async_copy(
        stage_v.at[(TR - 1) % 2], p_hbm.at[t0 + TR - 1, :, tb], osem).wait()


def kernel(x, tok_table, pos_emb):
    xT = x.astype(jnp.int32).T
    p = _embed(xT, tok_table, pos_emb[:T, :])
    return p.transpose(2, 4, 0, 1, 3).reshape(B, T, D)


# speedup vs baseline: 1.2267x; 1.2267x over previous
import functools

import jax
import jax.numpy as jnp
from jax import lax
from jax.experimental import pallas as pl
from jax.experimental.pallas import tpu as pltpu
from jax.experimental.pallas import tpu_sc as plsc

B, T, D = 1024, 200, 64
NC, NS = 2, 16
NW = NC * NS
BB = 128
NB = B // BB
TR = T // (NW // NB)
LANES = 16

_mesh = plsc.VectorSubcoreMesh(core_axis_name="c", subcore_axis_name="s")


@functools.partial(
    pl.kernel,
    out_type=jax.ShapeDtypeStruct((T, D // 8, NB, 8, BB), jnp.float32),
    mesh=_mesh,
    compiler_params=pltpu.CompilerParams(
        use_tc_tiling_on_sc=False, needs_layout_passes=False,
        disable_bounds_checks=True),
    scratch_types=[
        pltpu.VMEM((TR, BB), jnp.int32),
        pltpu.VMEM((2, BB, D), jnp.float32),
        pltpu.VMEM((2, D // 8, 8, BB), jnp.float32),
        pltpu.VMEM((T, D), jnp.float32),
        pltpu.SemaphoreType.DMA,
        pltpu.SemaphoreType.DMA,
    ],
)
def _embed(xT_hbm, tab_hbm, pos_hbm, p_hbm, idx_v, rows_v, stage_v,
           pos_v, gsem, osem):
    wid = lax.axis_index("s") * NC + lax.axis_index("c")
    tb = lax.rem(wid, NB)
    t0 = lax.div(wid, NB) * TR
    pltpu.sync_copy(pos_hbm, pos_v)
    pltpu.sync_copy(
        xT_hbm.at[pl.ds(t0, TR), pl.ds(tb * BB, BB)], idx_v)

    def start_gather(tl, buf):
        pltpu.async_copy(tab_hbm.at[idx_v.at[tl]], rows_v.at[buf], gsem)

    def wait_gather(tl, buf):
        pltpu.make_async_copy(
            tab_hbm.at[idx_v.at[tl]], rows_v.at[buf], gsem).wait()

    start_gather(0, 0)
    lanes = lax.iota(jnp.int32, LANES)

    def t_body(tl, carry):
        t = t0 + tl
        buf = lax.rem(tl, 2)
        nbuf = lax.rem(tl + 1, 2)

        @pl.when(tl >= 1)
        def _():
            pltpu.make_async_copy(
                stage_v.at[nbuf], p_hbm.at[t - 1, :, tb], osem).wait()

        @pl.when(tl + 1 < TR)
        def _():
            start_gather(tl + 1, nbuf)

        bufs = jnp.full((LANES,), buf, jnp.int32)
        for d0 in range(0, D, LANES):
            pos_row = pos_v[t, pl.ds(d0, LANES)]
            for j in range(LANES):
                pv = jax.lax.broadcast(pos_row[j], (LANES,))
                d = d0 + j
                for g in range(8):
                    stage_v[buf, d // 8, d % 8, pl.ds(g * LANES, LANES)] = pv

        wait_gather(tl, buf)

        for d0 in range(0, D, LANES):

            def _kbody(k, c2):
                rot = lax.rem(lanes + k, LANES)
                rot_hi = lax.shift_right_logical(rot, 3) + (d0 // 8)
                rot_lo = lax.bitwise_and(rot, 7)
                dcol = rot + d0
                for b0 in range(0, BB, LANES):
                    vec = plsc.load_gather(
                        rows_v, [bufs, lanes + b0, dcol])
                    plsc.addupdate_scatter(
                        stage_v, [bufs, rot_hi, rot_lo, lanes + b0], vec)
                return c2

            lax.fori_loop(0, LANES, _kbody, 0, unroll=2)

        pltpu.async_copy(stage_v.at[buf], p_hbm.at[t, :, tb], osem)
        return carry

    lax.fori_loop(0, TR, t_body, 0)
    pltpu.make_async_copy(
        stage_v.at[(TR - 1) % 2], p_hbm.at[t0 + TR - 1, :, tb], osem).wait()


def kernel(x, tok_table, pos_emb):
    xT = x.astype(jnp.int32).T
    p = _embed(xT, tok_table, pos_emb[:T, :])
    return p.transpose(2, 4, 0, 1, 3).reshape(B, T, D)


# speedup vs baseline: 1.2344x; 1.0063x over previous
import functools

import jax
import jax.numpy as jnp
from jax import lax
from jax.experimental import pallas as pl
from jax.experimental.pallas import tpu as pltpu
from jax.experimental.pallas import tpu_sc as plsc

B, T, D = 1024, 200, 64
NC, NS = 2, 16
NW = NC * NS
BB = 128
NB = B // BB
TR = T // (NW // NB)
LANES = 16

_mesh = plsc.VectorSubcoreMesh(core_axis_name="c", subcore_axis_name="s")


@functools.partial(
    pl.kernel,
    out_type=jax.ShapeDtypeStruct((T, D // 8, NB, 8, BB), jnp.float32),
    mesh=_mesh,
    compiler_params=pltpu.CompilerParams(
        use_tc_tiling_on_sc=False, needs_layout_passes=False,
        disable_bounds_checks=True),
    scratch_types=[
        pltpu.VMEM((TR, BB), jnp.int32),
        pltpu.VMEM((2, BB, D), jnp.float32),
        pltpu.VMEM((2, D // 8, 8, BB), jnp.float32),
        pltpu.VMEM((T, D), jnp.float32),
        pltpu.SemaphoreType.DMA,
        pltpu.SemaphoreType.DMA,
    ],
)
def _embed(xT_hbm, tab_hbm, pos_hbm, p_hbm, idx_v, rows_v, stage_v,
           pos_v, gsem, osem):
    wid = lax.axis_index("s") * NC + lax.axis_index("c")
    tb = lax.rem(wid, NB)
    t0 = lax.div(wid, NB) * TR
    pltpu.sync_copy(pos_hbm, pos_v)
    pltpu.sync_copy(
        xT_hbm.at[pl.ds(t0, TR), pl.ds(tb * BB, BB)], idx_v)

    def start_gather(tl, buf):
        pltpu.async_copy(tab_hbm.at[idx_v.at[tl]], rows_v.at[buf], gsem)

    def wait_gather(tl, buf):
        pltpu.make_async_copy(
            tab_hbm.at[idx_v.at[tl]], rows_v.at[buf], gsem).wait()

    start_gather(0, 0)
    lanes = lax.iota(jnp.int32, LANES)
    bvecs = [lanes + b0 for b0 in range(0, BB, LANES)]

    def t_body(tl, carry):
        t = t0 + tl
        buf = lax.rem(tl, 2)
        nbuf = lax.rem(tl + 1, 2)

        @pl.when(tl >= 1)
        def _():
            pltpu.make_async_copy(
                stage_v.at[nbuf], p_hbm.at[t - 1, :, tb], osem).wait()

        @pl.when(tl + 1 < TR)
        def _():
            start_gather(tl + 1, nbuf)

        for d0 in range(0, D, LANES):
            pos_row = pos_v[t, pl.ds(d0, LANES)]
            for j in range(LANES):
                pv = jax.lax.broadcast(pos_row[j], (LANES,))
                d = d0 + j
                for g in range(8):
                    stage_v[buf, d // 8, d % 8, pl.ds(g * LANES, LANES)] = pv

        wait_gather(tl, buf)

        for d0 in range(0, D, LANES):

            def _kbody(k, c2):
                rot = lax.rem(lanes + k, LANES)
                rot_hi = lax.shift_right_logical(rot, 3) + (d0 // 8)
                rot_lo = lax.bitwise_and(rot, 7)
                dcol = rot + d0
                rbuf = rows_v.at[buf]
                sbuf = stage_v.at[buf]
                for g in range(BB // LANES):
                    vec = plsc.load_gather(rbuf, [bvecs[g], dcol])
                    plsc.addupdate_scatter(
                        sbuf, [rot_hi, rot_lo, bvecs[g]], vec)
                return c2

            lax.fori_loop(0, LANES, _kbody, 0, unroll=2)

        pltpu.async_copy(stage_v.at[buf], p_hbm.at[t, :, tb], osem)
        return carry

    lax.fori_loop(0, TR, t_body, 0)
    pltpu.make_async_copy(
        stage_v.at[(TR - 1) % 2], p_hbm.at[t0 + TR - 1, :, tb], osem).wait()


def kernel(x, tok_table, pos_emb):
    xT = x.astype(jnp.int32).T
    p = _embed(xT, tok_table, pos_emb[:T, :])
    return p.transpose(2, 4, 0, 1, 3).reshape(B, T, D)
